# Initial kernel scaffold; baseline (speedup 1.0000x reference)
import jax
import jax.numpy as jnp
from jax.experimental import pallas as pl


def kernel(x, edge_index, edge_weight, edge_attr, batch_idx, emb_W, emb_b, fW1, fb1, fW2, fb2, iW1, ib1, iW2, ib2, bn_gamma, bn_beta, bn_mean, bn_var, oW1, ob1, oW2, ob2, oW3, ob3):
    raise NotImplementedError("write your pallas kernel here")


# speedup vs baseline: 2.7520x; 2.7520x over previous
import functools

import jax
import jax.numpy as jnp
import numpy as np
from jax import lax
from jax.experimental import pallas as pl
from jax.experimental.pallas import tpu as pltpu
from jax.experimental.pallas import tpu_sc as plsc

N = 10000
E = 160000
DF = 128
H = 64
F = 32
NI = 3
G = 64
CUT = 8.0

NC = 2
NS = 16
NW = NC * NS
CHUNK = 128
E_PAD = 163840
CPW = E_PAD // (NW * CHUNK)
NPAD = 10240
ROWS_PER_TILE = NPAD // NS


def _embed_body(x_ref, w_ref, b_ref, o_ref):
    o_ref[...] = (
        jnp.dot(x_ref[...], w_ref[...], preferred_element_type=jnp.float32)
        + b_ref[...]
    )


def _embed(x, emb_W, emb_b):
    blk = 1000
    return pl.pallas_call(
        _embed_body,
        grid=(N // blk,),
        in_specs=[
            pl.BlockSpec((blk, DF), lambda i: (i, 0)),
            pl.BlockSpec((DF, H), lambda i: (0, 0)),
            pl.BlockSpec((1, H), lambda i: (0, 0)),
        ],
        out_specs=pl.BlockSpec((blk, H), lambda i: (i, 0)),
        out_shape=jax.ShapeDtypeStruct((N, H), jnp.float32),
    )(x, emb_W, emb_b.reshape(1, H))


def _edge_scalar_body(ew_ref, w1_ref, b1_ref, w2_ref, b2_ref, o_ref):
    ew = ew_ref[0]
    scaled = ew * (2.0 / CUT) - 1.0
    cut = jnp.where(ew <= CUT, 0.5 * (jnp.cos(ew * (np.pi / CUT)) + 1.0), 0.0)
    for i in range(NI):
        f = jnp.tanh(w1_ref[i] * scaled + b1_ref[i])
        f2 = jnp.dot(w2_ref[i], f, preferred_element_type=jnp.float32) + b2_ref[i]
        o_ref[i, 0] = jnp.sum(f2 * cut, axis=0, keepdims=True)


EB = 4096


def _edge_scalars(ew_pad, w1T, b1T, w2T, b2T):
    rows = E_PAD // EB
    return pl.pallas_call(
        _edge_scalar_body,
        grid=(rows,),
        in_specs=[
            pl.BlockSpec((1, 1, EB), lambda i: (i, 0, 0)),
            pl.BlockSpec((NI, F, 1), lambda i: (0, 0, 0)),
            pl.BlockSpec((NI, F, 1), lambda i: (0, 0, 0)),
            pl.BlockSpec((NI, F, F), lambda i: (0, 0, 0)),
            pl.BlockSpec((NI, F, 1), lambda i: (0, 0, 0)),
        ],
        out_specs=pl.BlockSpec((NI, 1, 1, EB), lambda i: (0, i, 0, 0)),
        out_shape=jax.ShapeDtypeStruct((NI, rows, 1, EB), jnp.float32),
    )(ew_pad, w1T, b1T, w2T, b2T)


def _softplus(x):
    return jnp.maximum(x, 0.0) + jnp.log1p(jnp.exp(-jnp.abs(x)))


def _interact_body(aggA_ref, aggB_ref, h_ref, w1_ref, b1_ref, w2_ref, b2_ref,
                   g_ref, be_ref, mu_ref, var_ref, o_ref):
    agg = aggA_ref[...] + aggB_ref[...]
    t = _softplus(
        jnp.dot(agg, w1_ref[...], preferred_element_type=jnp.float32) + b1_ref[...]
    )
    y = jnp.dot(t, w2_ref[...], preferred_element_type=jnp.float32) + b2_ref[...]
    y = g_ref[...] * (y - mu_ref[...]) / jnp.sqrt(var_ref[...] + 1e-3) + be_ref[...]
    o_ref[...] = h_ref[...] + y


def _interact(aggA, aggB, h, iW1, ib1, iW2, ib2, g, be, mu, var):
    blk = 1000
    vec = pl.BlockSpec((1, H), lambda i: (0, 0))
    return pl.pallas_call(
        _interact_body,
        grid=(N // blk,),
        in_specs=[
            pl.BlockSpec((blk, H), lambda i: (i, 0)),
            pl.BlockSpec((blk, H), lambda i: (i, 0)),
            pl.BlockSpec((blk, H), lambda i: (i, 0)),
            pl.BlockSpec((H, H), lambda i: (0, 0)),
            vec,
            pl.BlockSpec((H, H), lambda i: (0, 0)),
            vec, vec, vec, vec, vec,
        ],
        out_specs=pl.BlockSpec((blk, H), lambda i: (i, 0)),
        out_shape=jax.ShapeDtypeStruct((N, H), jnp.float32),
    )(aggA, aggB, h, iW1, ib1.reshape(1, H), iW2, ib2.reshape(1, H),
      g.reshape(1, H), be.reshape(1, H), mu.reshape(1, H), var.reshape(1, H))


def _pool_body(batch_ref, h_ref, o_ref):
    i = pl.program_id(0)
    bi = batch_ref[0, 0, :]
    gids = lax.broadcasted_iota(jnp.int32, (G, bi.shape[0]), 0)
    P = (gids == bi[None, :]).astype(jnp.float32)
    part = lax.dot_general(P, h_ref[...], (((1,), (0,)), ((), ())),
                           precision=lax.Precision.HIGHEST,
                           preferred_element_type=jnp.float32)

    @pl.when(i == 0)
    def _():
        o_ref[...] = jnp.zeros_like(o_ref)

    o_ref[...] += part


def _pool(h, batch_idx):
    blk = 1000
    nb = N // blk
    batch3 = batch_idx.reshape(nb, 1, blk)
    return pl.pallas_call(
        _pool_body,
        grid=(nb,),
        in_specs=[
            pl.BlockSpec((1, 1, blk), lambda i: (i, 0, 0)),
            pl.BlockSpec((blk, H), lambda i: (i, 0)),
        ],
        out_specs=pl.BlockSpec((G, H), lambda i: (0, 0)),
        out_shape=jax.ShapeDtypeStruct((G, H), jnp.float32),
    )(batch3, h)


def _outmlp_body(p_ref, w1_ref, b1_ref, w2_ref, b2_ref, w3_ref, b3_ref, o_ref):
    o = _softplus(
        jnp.dot(p_ref[...], w1_ref[...], preferred_element_type=jnp.float32)
        + b1_ref[...]
    )
    o = _softplus(
        jnp.dot(o, w2_ref[...], preferred_element_type=jnp.float32) + b2_ref[...]
    )
    o_ref[...] = (
        jnp.dot(o, w3_ref[...], preferred_element_type=jnp.float32) + b3_ref[...]
    )


def _outmlp(pooled, oW1, ob1, oW2, ob2, oW3, ob3):
    return pl.pallas_call(
        _outmlp_body,
        out_shape=jax.ShapeDtypeStruct((G, 1), jnp.float32),
    )(pooled, oW1, ob1.reshape(1, H // 2), oW2, ob2.reshape(1, H // 2),
      oW3, ob3.reshape(1, 1))


def _sc_edge_body(h_hbm, col_hbm, row_hbm, s_hbm, zero_hbm, out_hbm,
                  col_v, row_v, s_v, rows_v, acc_sh, gsem, ssem):
    cid = lax.axis_index("c")
    sid = lax.axis_index("s")
    wid = cid * NS + sid

    pltpu.sync_copy(zero_hbm.at[pl.ds(sid * ROWS_PER_TILE, ROWS_PER_TILE)],
                    acc_sh.at[pl.ds(sid * ROWS_PER_TILE, ROWS_PER_TILE)])

    pltpu.sync_copy(col_hbm.at[pl.ds(wid * CPW, CPW)], col_v)
    pltpu.sync_copy(row_hbm.at[pl.ds(wid * CPW, CPW)], row_v)
    pltpu.sync_copy(s_hbm.at[pl.ds(wid * CPW, CPW)], s_v)

    plsc.subcore_barrier()

    def chunk_body(j, carry):
        pltpu.async_copy(h_hbm.at[col_v.at[j]], rows_v, gsem).wait()

        def scale_body(g, c2):
            sv16 = s_v[j, pl.ds(g * 16, 16)]
            base = g * 16
            for t in range(16):
                sval = sv16[t]
                for q in range(H // 16):
                    sl = pl.ds(q * 16, 16)
                    rows_v[base + t, sl] = rows_v[base + t, sl] * sval
            return c2

        lax.fori_loop(0, CHUNK // 16, scale_body, 0)

        pltpu.async_copy(rows_v, acc_sh.at[row_v.at[j]], ssem, add=True).wait()
        return carry

    lax.fori_loop(0, CPW, chunk_body, 0)

    plsc.subcore_barrier()

    pltpu.sync_copy(acc_sh.at[pl.ds(sid * ROWS_PER_TILE, ROWS_PER_TILE)],
                    out_hbm.at[cid, pl.ds(sid * ROWS_PER_TILE, ROWS_PER_TILE)])


def _sc_edge(h, col2d, row2d, s2d, zeros_nh):
    mesh = plsc.VectorSubcoreMesh(core_axis_name="c", subcore_axis_name="s")
    kern = pl.kernel(
        _sc_edge_body,
        out_type=jax.ShapeDtypeStruct((NC, NPAD, H), jnp.float32),
        mesh=mesh,
        compiler_params=pltpu.CompilerParams(use_tc_tiling_on_sc=False),
        scratch_types=[
            pltpu.VMEM((CPW, CHUNK), jnp.int32),
            pltpu.VMEM((CPW, CHUNK), jnp.int32),
            pltpu.VMEM((CPW, CHUNK), jnp.float32),
            pltpu.VMEM((CHUNK, H), jnp.float32),
            pltpu.VMEM_SHARED((NPAD, H), jnp.float32),
            pltpu.SemaphoreType.DMA,
            pltpu.SemaphoreType.DMA,
        ],
    )
    return kern(h, col2d, row2d, s2d, zeros_nh)


def kernel(x, edge_index, edge_weight, edge_attr, batch_idx, emb_W, emb_b,
           fW1, fb1, fW2, fb2, iW1, ib1, iW2, ib2, bn_gamma, bn_beta,
           bn_mean, bn_var, oW1, ob1, oW2, ob2, oW3, ob3):
    w1T = jnp.transpose(fW1, (0, 2, 1))
    b1T = fb1[:, :, None]
    w2T = jnp.transpose(fW2, (0, 2, 1))
    b2T = fb2[:, :, None]

    pad = E_PAD - E
    col_p = jnp.concatenate([edge_index[1], jnp.zeros((pad,), jnp.int32)])
    row_p = jnp.concatenate([edge_index[0], jnp.zeros((pad,), jnp.int32)])
    ew_p = jnp.concatenate([edge_weight,
                            jnp.full((pad,), 2.0 * CUT, jnp.float32)])
    col2d = col_p.reshape(E_PAD // CHUNK, CHUNK)
    row2d = row_p.reshape(E_PAD // CHUNK, CHUNK)
    zeros_nh = jnp.zeros((NPAD, H), jnp.float32)

    S = _edge_scalars(ew_p.reshape(E_PAD // EB, 1, EB), w1T, b1T, w2T, b2T)

    h = _embed(x, emb_W, emb_b)

    for i in range(NI):
        s2d = S[i].reshape(E_PAD // CHUNK, CHUNK)
        agg2 = _sc_edge(h, col2d, row2d, s2d, zeros_nh)[:, :N, :]
        h = _interact(agg2[0], agg2[1], h, iW1[i], ib1[i], iW2[i], ib2[i],
                      bn_gamma[i], bn_beta[i], bn_mean[i], bn_var[i])

    pooled = _pool(h, batch_idx)
    o = _outmlp(pooled, oW1, ob1, oW2, ob2, oW3, ob3)
    return jnp.squeeze(o, -1)


# speedup vs baseline: 4.2547x; 1.5460x over previous
import functools

import jax
import jax.numpy as jnp
import numpy as np
from jax import lax
from jax.experimental import pallas as pl
from jax.experimental.pallas import tpu as pltpu
from jax.experimental.pallas import tpu_sc as plsc

N = 10000
E = 160000
DF = 128
H = 64
F = 32
NI = 3
G = 64
CUT = 8.0

NC = 2
NS = 16
NW = NC * NS
CHUNK = 128
E_PAD = 163840
CPW = E_PAD // (NW * CHUNK)
NPAD = 10240
ROWS_PER_TILE = NPAD // NS


def _embed_body(x_ref, w_ref, b_ref, o_ref):
    o_ref[...] = (
        jnp.dot(x_ref[...], w_ref[...], preferred_element_type=jnp.float32)
        + b_ref[...]
    )


def _embed(x, emb_W, emb_b):
    blk = 1000
    return pl.pallas_call(
        _embed_body,
        grid=(N // blk,),
        in_specs=[
            pl.BlockSpec((blk, DF), lambda i: (i, 0)),
            pl.BlockSpec((DF, H), lambda i: (0, 0)),
            pl.BlockSpec((1, H), lambda i: (0, 0)),
        ],
        out_specs=pl.BlockSpec((blk, H), lambda i: (i, 0)),
        out_shape=jax.ShapeDtypeStruct((N, H), jnp.float32),
    )(x, emb_W, emb_b.reshape(1, H))


def _edge_scalar_body(ew_ref, w1_ref, b1_ref, w2_ref, b2_ref, o_ref):
    ew = ew_ref[0]
    scaled = ew * (2.0 / CUT) - 1.0
    cut = jnp.where(ew <= CUT, 0.5 * (jnp.cos(ew * (np.pi / CUT)) + 1.0), 0.0)
    for i in range(NI):
        f = jnp.tanh(w1_ref[i] * scaled + b1_ref[i])
        f2 = jnp.dot(w2_ref[i], f, preferred_element_type=jnp.float32) + b2_ref[i]
        o_ref[i, 0] = jnp.sum(f2 * cut, axis=0, keepdims=True)


EB = 4096


def _edge_scalars(ew_pad, w1T, b1T, w2T, b2T):
    rows = E_PAD // EB
    return pl.pallas_call(
        _edge_scalar_body,
        grid=(rows,),
        in_specs=[
            pl.BlockSpec((1, 1, EB), lambda i: (i, 0, 0)),
            pl.BlockSpec((NI, F, 1), lambda i: (0, 0, 0)),
            pl.BlockSpec((NI, F, 1), lambda i: (0, 0, 0)),
            pl.BlockSpec((NI, F, F), lambda i: (0, 0, 0)),
            pl.BlockSpec((NI, F, 1), lambda i: (0, 0, 0)),
        ],
        out_specs=pl.BlockSpec((NI, 1, 1, EB), lambda i: (0, i, 0, 0)),
        out_shape=jax.ShapeDtypeStruct((NI, rows, 1, EB), jnp.float32),
    )(ew_pad, w1T, b1T, w2T, b2T)


def _softplus(x):
    return jnp.maximum(x, 0.0) + jnp.log1p(jnp.exp(-jnp.abs(x)))


def _interact_body(aggA_ref, aggB_ref, h_ref, w1_ref, b1_ref, w2_ref, b2_ref,
                   g_ref, be_ref, mu_ref, var_ref, o_ref):
    agg = aggA_ref[...] + aggB_ref[...]
    t = _softplus(
        jnp.dot(agg, w1_ref[...], preferred_element_type=jnp.float32) + b1_ref[...]
    )
    y = jnp.dot(t, w2_ref[...], preferred_element_type=jnp.float32) + b2_ref[...]
    y = g_ref[...] * (y - mu_ref[...]) / jnp.sqrt(var_ref[...] + 1e-3) + be_ref[...]
    o_ref[...] = h_ref[...] + y


def _interact(aggA, aggB, h, iW1, ib1, iW2, ib2, g, be, mu, var):
    blk = 1000
    vec = pl.BlockSpec((1, H), lambda i: (0, 0))
    return pl.pallas_call(
        _interact_body,
        grid=(N // blk,),
        in_specs=[
            pl.BlockSpec((blk, H), lambda i: (i, 0)),
            pl.BlockSpec((blk, H), lambda i: (i, 0)),
            pl.BlockSpec((blk, H), lambda i: (i, 0)),
            pl.BlockSpec((H, H), lambda i: (0, 0)),
            vec,
            pl.BlockSpec((H, H), lambda i: (0, 0)),
            vec, vec, vec, vec, vec,
        ],
        out_specs=pl.BlockSpec((blk, H), lambda i: (i, 0)),
        out_shape=jax.ShapeDtypeStruct((N, H), jnp.float32),
    )(aggA, aggB, h, iW1, ib1.reshape(1, H), iW2, ib2.reshape(1, H),
      g.reshape(1, H), be.reshape(1, H), mu.reshape(1, H), var.reshape(1, H))


def _pool_body(batch_ref, h_ref, o_ref):
    i = pl.program_id(0)
    bi = batch_ref[0, 0, :]
    gids = lax.broadcasted_iota(jnp.int32, (G, bi.shape[0]), 0)
    P = (gids == bi[None, :]).astype(jnp.float32)
    part = lax.dot_general(P, h_ref[...], (((1,), (0,)), ((), ())),
                           precision=lax.Precision.HIGHEST,
                           preferred_element_type=jnp.float32)

    @pl.when(i == 0)
    def _():
        o_ref[...] = jnp.zeros_like(o_ref)

    o_ref[...] += part


def _pool(h, batch_idx):
    blk = 1000
    nb = N // blk
    batch3 = batch_idx.reshape(nb, 1, blk)
    return pl.pallas_call(
        _pool_body,
        grid=(nb,),
        in_specs=[
            pl.BlockSpec((1, 1, blk), lambda i: (i, 0, 0)),
            pl.BlockSpec((blk, H), lambda i: (i, 0)),
        ],
        out_specs=pl.BlockSpec((G, H), lambda i: (0, 0)),
        out_shape=jax.ShapeDtypeStruct((G, H), jnp.float32),
    )(batch3, h)


def _outmlp_body(p_ref, w1_ref, b1_ref, w2_ref, b2_ref, w3_ref, b3_ref, o_ref):
    o = _softplus(
        jnp.dot(p_ref[...], w1_ref[...], preferred_element_type=jnp.float32)
        + b1_ref[...]
    )
    o = _softplus(
        jnp.dot(o, w2_ref[...], preferred_element_type=jnp.float32) + b2_ref[...]
    )
    o_ref[...] = (
        jnp.dot(o, w3_ref[...], preferred_element_type=jnp.float32) + b3_ref[...]
    )


def _outmlp(pooled, oW1, ob1, oW2, ob2, oW3, ob3):
    return pl.pallas_call(
        _outmlp_body,
        out_shape=jax.ShapeDtypeStruct((G, 1), jnp.float32),
    )(pooled, oW1, ob1.reshape(1, H // 2), oW2, ob2.reshape(1, H // 2),
      oW3, ob3.reshape(1, 1))


def _sc_edge_body(h_hbm, col_hbm, row_hbm, s_hbm, zero_hbm, out_hbm,
                  col_v, row_v, s_v, g0, g1, sb0, sb1, acc_sh,
                  gsem0, gsem1, ssem0, ssem1):
    cid = lax.axis_index("c")
    sid = lax.axis_index("s")
    wid = cid * NS + sid

    pltpu.sync_copy(zero_hbm.at[pl.ds(sid * ROWS_PER_TILE, ROWS_PER_TILE)],
                    acc_sh.at[pl.ds(sid * ROWS_PER_TILE, ROWS_PER_TILE)])
    pltpu.sync_copy(col_hbm.at[pl.ds(wid * CPW, CPW)], col_v)
    pltpu.sync_copy(row_hbm.at[pl.ds(wid * CPW, CPW)], row_v)
    pltpu.sync_copy(s_hbm.at[pl.ds(wid * CPW, CPW)], s_v)
    plsc.subcore_barrier()

    gbuf = (g0, g1)
    sbuf = (sb0, sb1)
    gsem = (gsem0, gsem1)
    ssem = (ssem0, ssem1)

    def start_gather(j, p):
        pltpu.async_copy(h_hbm.at[col_v.at[j]], gbuf[p], gsem[p])

    def wait_gather(j, p):
        pltpu.make_async_copy(h_hbm.at[col_v.at[j]], gbuf[p], gsem[p]).wait()

    def start_scatter(j, p):
        pltpu.async_copy(sbuf[p], acc_sh.at[row_v.at[j]], ssem[p], add=True)

    def wait_scatter(j, p):
        pltpu.make_async_copy(sbuf[p], acc_sh.at[row_v.at[j]], ssem[p]).wait()

    start_gather(0, 0)
    start_gather(1, 1)

    def pair_body(k, carry):
        for p in range(2):
            j = 2 * k + p
            wait_gather(j, p)

            @pl.when(k >= 1)
            def _():
                wait_scatter(j - 2, p)

            def scale_body(g, c2):
                sv16 = s_v[j, pl.ds(g * 16, 16)]
                base = g * 16
                for t in range(16):
                    sval = sv16[t]
                    for q in range(H // 16):
                        sl = pl.ds(q * 16, 16)
                        sbuf[p][base + t, sl] = gbuf[p][base + t, sl] * sval
                return c2

            lax.fori_loop(0, CHUNK // 16, scale_body, 0)
            start_scatter(j, p)

            @pl.when(k < CPW // 2 - 1)
            def _():
                start_gather(j + 2, p)
        return carry

    lax.fori_loop(0, CPW // 2, pair_body, 0)
    wait_scatter(CPW - 2, 0)
    wait_scatter(CPW - 1, 1)

    plsc.subcore_barrier()

    pltpu.sync_copy(acc_sh.at[pl.ds(sid * ROWS_PER_TILE, ROWS_PER_TILE)],
                    out_hbm.at[cid, pl.ds(sid * ROWS_PER_TILE, ROWS_PER_TILE)])


def _sc_edge(h, col2d, row2d, s2d, zeros_nh):
    mesh = plsc.VectorSubcoreMesh(core_axis_name="c", subcore_axis_name="s")
    kern = pl.kernel(
        _sc_edge_body,
        out_type=jax.ShapeDtypeStruct((NC, NPAD, H), jnp.float32),
        mesh=mesh,
        compiler_params=pltpu.CompilerParams(use_tc_tiling_on_sc=False),
        scratch_types=[
            pltpu.VMEM((CPW, CHUNK), jnp.int32),
            pltpu.VMEM((CPW, CHUNK), jnp.int32),
            pltpu.VMEM((CPW, CHUNK), jnp.float32),
            pltpu.VMEM((CHUNK, H), jnp.float32),
            pltpu.VMEM((CHUNK, H), jnp.float32),
            pltpu.VMEM((CHUNK, H), jnp.float32),
            pltpu.VMEM((CHUNK, H), jnp.float32),
            pltpu.VMEM_SHARED((NPAD, H), jnp.float32),
            pltpu.SemaphoreType.DMA,
            pltpu.SemaphoreType.DMA,
            pltpu.SemaphoreType.DMA,
            pltpu.SemaphoreType.DMA,
        ],
    )
    return kern(h, col2d, row2d, s2d, zeros_nh)


def kernel(x, edge_index, edge_weight, edge_attr, batch_idx, emb_W, emb_b,
           fW1, fb1, fW2, fb2, iW1, ib1, iW2, ib2, bn_gamma, bn_beta,
           bn_mean, bn_var, oW1, ob1, oW2, ob2, oW3, ob3):
    w1T = jnp.transpose(fW1, (0, 2, 1))
    b1T = fb1[:, :, None]
    w2T = jnp.transpose(fW2, (0, 2, 1))
    b2T = fb2[:, :, None]

    pad = E_PAD - E
    col_p = jnp.concatenate([edge_index[1], jnp.zeros((pad,), jnp.int32)])
    row_p = jnp.concatenate([edge_index[0], jnp.zeros((pad,), jnp.int32)])
    ew_p = jnp.concatenate([edge_weight,
                            jnp.full((pad,), 2.0 * CUT, jnp.float32)])
    col2d = col_p.reshape(E_PAD // CHUNK, CHUNK)
    row2d = row_p.reshape(E_PAD // CHUNK, CHUNK)
    zeros_nh = jnp.zeros((NPAD, H), jnp.float32)

    S = _edge_scalars(ew_p.reshape(E_PAD // EB, 1, EB), w1T, b1T, w2T, b2T)

    h = _embed(x, emb_W, emb_b)

    for i in range(NI):
        s2d = S[i].reshape(E_PAD // CHUNK, CHUNK)
        agg2 = _sc_edge(h, col2d, row2d, s2d, zeros_nh)[:, :N, :]
        h = _interact(agg2[0], agg2[1], h, iW1[i], ib1[i], iW2[i], ib2[i],
                      bn_gamma[i], bn_beta[i], bn_mean[i], bn_var[i])

    pooled = _pool(h, batch_idx)
    o = _outmlp(pooled, oW1, ob1, oW2, ob2, oW3, ob3)
    return jnp.squeeze(o, -1)
